# baseline (speedup 1.0000x reference)
import jax
import jax.numpy as jnp
from jax import lax
from jax.experimental import pallas as pl
from jax.experimental.pallas import tpu as pltpu
from jax.experimental.pallas import tpu_sc as plsc

_NC = 2
_NS = 16
_NW = _NC * _NS
_CHUNK = 128
_NCH = 4
_RPW = _CHUNK * _NCH
_K = 4
_ES = 32
_L = 16


def _fused_body(bi_flat, ent_hbm, rel4_hbm, w_hbm,
                norm_out, att_out,
                raw_v, idxh_v, idxr_v, idxt_v, mod32_v,
                h0, h1, t0, t1, r0, r1,
                norm_v, att_v, w_v,
                sem_h0, sem_h1, sem_t0, sem_t1, sem_r0, sem_r1):
    wid = lax.axis_index("s") * _NC + lax.axis_index("c")
    base_row = wid * _RPW
    pltpu.sync_copy(w_hbm, w_v)
    iota = lax.iota(jnp.int32, _L)

    def wbc(j):
        return plsc.load_gather(w_v, [jnp.full((_L,), j, jnp.int32)])

    for ch in range(_NCH):
        pltpu.sync_copy(
            bi_flat.at[pl.ds((base_row + ch * _CHUNK) * 3, _CHUNK * 3)], raw_v)
        for g in range(_CHUNK // _L):
            fbase = iota * 3 + g * (3 * _L)
            hvec = plsc.load_gather(raw_v, [fbase])
            rvec = plsc.load_gather(raw_v, [fbase + 1])
            tvec = plsc.load_gather(raw_v, [fbase + 2])
            sl = pl.ds(g * _L, _L)
            idxh_v[ch, sl] = hvec
            idxr_v[ch, sl] = rvec >> 2
            idxt_v[ch, sl] = tvec
            mod32_v[ch, sl] = (rvec & 3) * _ES

    h_bufs, t_bufs, r_bufs = (h0, h1), (t0, t1), (r0, r1)
    sem_h, sem_t, sem_r = (sem_h0, sem_h1), (sem_t0, sem_t1), (sem_r0, sem_r1)

    def issue(ch):
        p = ch % 2
        return (
            pltpu.async_copy(ent_hbm.at[idxh_v.at[ch]], h_bufs[p], sem_h[p]),
            pltpu.async_copy(ent_hbm.at[idxt_v.at[ch]], t_bufs[p], sem_t[p]),
            pltpu.async_copy(rel4_hbm.at[idxr_v.at[ch]], r_bufs[p], sem_r[p]),
        )

    bias = wbc(3 * _ES)
    pending = issue(0)

    for ch in range(_NCH):
        p = ch % 2
        for cp in pending:
            cp.wait()
        if ch + 1 < _NCH:
            pending = issue(ch + 1)
        hb, tb, rb = h_bufs[p], t_bufs[p], r_bufs[p]

        def group_body(g, _):
            row = iota + g * _L
            mod32 = plsc.load_gather(mod32_v, [jnp.full((_L,), ch, jnp.int32), row])
            zero = jnp.zeros((_L,), jnp.float32)

            def dots_body(d8, carry):
                a0, a1, a2, a3, ra = carry
                for i in range(4):
                    d = d8 * 4 + i
                    rc = plsc.load_gather(rb, [row, mod32 + d])
                    ra = ra + rc * wbc(_ES + d)
                    accs = [a0, a1, a2, a3]
                    for k in range(_K):
                        col = jnp.full((_L,), k * _ES, jnp.int32) + d
                        hc = plsc.load_gather(hb, [row, col])
                        tc = plsc.load_gather(tb, [row, col])
                        accs[k] = accs[k] + hc * wbc(d) + tc * wbc(2 * _ES + d)
                    a0, a1, a2, a3 = accs
                return a0, a1, a2, a3, ra

            a0, a1, a2, a3, ra = lax.fori_loop(
                0, _ES // 4, dots_body, (zero, zero, zero, zero, zero))
            t_list = [jnp.maximum(a + ra + bias, 0.0) for a in (a0, a1, a2, a3)]
            m = jnp.maximum(jnp.maximum(t_list[0], t_list[1]),
                            jnp.maximum(t_list[2], t_list[3]))
            e_list = [jnp.exp(tv - m) for tv in t_list]
            inv = 1.0 / (e_list[0] + e_list[1] + e_list[2] + e_list[3])
            att = [e * inv for e in e_list]
            arow = (row + ch * _CHUNK) * _K
            for k in range(_K):
                plsc.store_scatter(att_v, [arow + k], att[k])

            def comb_body(d8, nacc):
                for i in range(4):
                    d = d8 * 4 + i
                    v = plsc.load_gather(rb, [row, mod32 + d])
                    for k in range(_K):
                        col = jnp.full((_L,), k * _ES, jnp.int32) + d
                        hc = plsc.load_gather(hb, [row, col])
                        tc = plsc.load_gather(tb, [row, col])
                        v = v + att[k] * (hc - tc)
                    nacc = nacc + jnp.abs(v)
                return nacc

            norm = lax.fori_loop(0, _ES // 4, comb_body, zero)
            plsc.store_scatter(norm_v, [row + ch * _CHUNK], norm)
            return 0

        lax.fori_loop(0, _CHUNK // _L, group_body, 0)

    pltpu.sync_copy(norm_v, norm_out.at[pl.ds(base_row, _RPW)])
    pltpu.sync_copy(att_v, att_out.at[pl.ds(base_row * _K, _RPW * _K)])


def _sc_fused(bi_flat, entity_emb, rel4, wb):
    b = _NW * _RPW
    mesh = plsc.VectorSubcoreMesh(core_axis_name="c", subcore_axis_name="s",
                                  num_cores=_NC, num_subcores=_NS)
    return pl.kernel(
        _fused_body,
        out_type=(
            jax.ShapeDtypeStruct((b,), jnp.float32),
            jax.ShapeDtypeStruct((b * _K,), jnp.float32),
        ),
        mesh=mesh,
        scratch_types=[
            pltpu.VMEM((_CHUNK * 3,), jnp.int32),
            pltpu.VMEM((_NCH, _CHUNK), jnp.int32),
            pltpu.VMEM((_NCH, _CHUNK), jnp.int32),
            pltpu.VMEM((_NCH, _CHUNK), jnp.int32),
            pltpu.VMEM((_NCH, _CHUNK), jnp.int32),
            pltpu.VMEM((_CHUNK, 128), jnp.float32),
            pltpu.VMEM((_CHUNK, 128), jnp.float32),
            pltpu.VMEM((_CHUNK, 128), jnp.float32),
            pltpu.VMEM((_CHUNK, 128), jnp.float32),
            pltpu.VMEM((_CHUNK, 128), jnp.float32),
            pltpu.VMEM((_CHUNK, 128), jnp.float32),
            pltpu.VMEM((_RPW,), jnp.float32),
            pltpu.VMEM((_RPW * _K,), jnp.float32),
            pltpu.VMEM((3 * _ES + 1,), jnp.float32),
            pltpu.SemaphoreType.DMA,
            pltpu.SemaphoreType.DMA,
            pltpu.SemaphoreType.DMA,
            pltpu.SemaphoreType.DMA,
            pltpu.SemaphoreType.DMA,
            pltpu.SemaphoreType.DMA,
        ],
        compiler_params=pltpu.CompilerParams(needs_layout_passes=False),
    )(bi_flat, entity_emb, rel4, wb)


def kernel(batch_inputs, entity_emb, relation_emb, fc1_w, fc1_b):
    b = batch_inputs.shape[0]
    rel4 = relation_emb.reshape(
        relation_emb.shape[0] * relation_emb.shape[1] // 128, 128)
    wb = jnp.concatenate([fc1_w.reshape(-1), fc1_b.reshape(-1)])
    norm, att_flat = _sc_fused(batch_inputs.reshape(-1), entity_emb, rel4, wb)
    att = att_flat.reshape(b, _K)
    len_pos = b // 4
    pos_norm = jnp.tile(norm[:len_pos], (3,))
    neg_norm = norm[len_pos:]
    y = jnp.full((3 * len_pos,), -1.0, dtype=jnp.float32)
    return (pos_norm, neg_norm, y, att)
